# Initial kernel scaffold; baseline (speedup 1.0000x reference)
import functools

import jax
import jax.numpy as jnp
from jax import lax
from jax.experimental import pallas as pl
from jax.experimental.pallas import tpu as pltpu
from jax.experimental.pallas import tpu_sc as plsc

N = 10000
D = 128
NP = 10240
NC = 2
NS = 16
NW = NC * NS
CH = 128
RPS = NP // NS

_MESH = plsc.VectorSubcoreMesh(core_axis_name="c", subcore_axis_name="s")


def _worker_id():
    return lax.axis_index("s") * NC + lax.axis_index("c")


def _fill_rows(ref, rows, value):
    cols = ref.shape[1]

    @pl.loop(0, rows)
    def _(r):
        @pl.loop(0, cols // 16)
        def _(cc):
            ref[r, pl.ds(cc * 16, 16)] = jnp.full((16,), value, jnp.float32)


def _zero_acc(acc_sh, zrow_v):
    sid = lax.axis_index("s")
    base = sid * RPS
    zr = zrow_v.shape[0]

    @pl.loop(0, RPS // zr)
    def _(t):
        pltpu.sync_copy(zrow_v, acc_sh.at[pl.ds(base + t * zr, zr)])


def _make_sc_deg(K):
    @functools.partial(
        pl.kernel,
        mesh=_MESH,
        out_type=jax.ShapeDtypeStruct((NC, NP, 16), jnp.float32),
        scratch_types=[
            pltpu.VMEM((K, CH), jnp.int32),
            pltpu.VMEM((CH, 16), jnp.float32),
            pltpu.VMEM((128, 16), jnp.float32),
            pltpu.VMEM_SHARED((NP, 16), jnp.float32),
        ],
    )
    def sc_deg(dst_hbm, out_hbm, idx_v, ones_v, zrow_v, acc_sh):
        cid = lax.axis_index("c")
        sid = lax.axis_index("s")
        wid = _worker_id()
        base = sid * RPS

        _fill_rows(ones_v, CH, 1.0)
        _fill_rows(zrow_v, 128, 0.0)
        _zero_acc(acc_sh, zrow_v)
        plsc.subcore_barrier()

        pltpu.sync_copy(dst_hbm.at[wid], idx_v)

        @pl.loop(0, K)
        def _(j):
            pltpu.sync_copy(ones_v, acc_sh.at[idx_v.at[j]], add=True)

        plsc.subcore_barrier()
        pltpu.sync_copy(
            acc_sh.at[pl.ds(base, RPS)], out_hbm.at[cid].at[pl.ds(base, RPS)]
        )

    return sc_deg


def _make_sc_edge(K):
    @functools.partial(
        pl.kernel,
        mesh=_MESH,
        out_type=jax.ShapeDtypeStruct((NC, NP, D), jnp.float32),
        scratch_types=[
            pltpu.VMEM((K, CH), jnp.int32),
            pltpu.VMEM((K, CH), jnp.int32),
            pltpu.VMEM((CH, D), jnp.float32),
            pltpu.VMEM((128, D), jnp.float32),
            pltpu.VMEM_SHARED((NP, D), jnp.float32),
        ],
    )
    def sc_edge(tab_hbm, src_hbm, dst_hbm, out_hbm, si_v, di_v, rows_v, zrow_v,
                acc_sh):
        cid = lax.axis_index("c")
        sid = lax.axis_index("s")
        wid = _worker_id()
        base = sid * RPS

        _fill_rows(zrow_v, 128, 0.0)
        _zero_acc(acc_sh, zrow_v)
        plsc.subcore_barrier()

        pltpu.sync_copy(src_hbm.at[wid], si_v)
        pltpu.sync_copy(dst_hbm.at[wid], di_v)

        @pl.loop(0, K)
        def _(j):
            pltpu.sync_copy(tab_hbm.at[si_v.at[j]], rows_v)
            pltpu.sync_copy(rows_v, acc_sh.at[di_v.at[j]], add=True)

        plsc.subcore_barrier()
        pltpu.sync_copy(
            acc_sh.at[pl.ds(base, RPS)], out_hbm.at[cid].at[pl.ds(base, RPS)]
        )

    return sc_edge


def _dinv_from_parts(degp_ref):
    deg = 1.0 + degp_ref[0, 0:N, 0:1] + degp_ref[1, 0:N, 0:1]
    return lax.rsqrt(deg)


def _tc1_body(x_ref, w_ref, degp_ref, hp_ref):
    dinv = _dinv_from_parts(degp_ref)
    h = lax.dot_general(
        x_ref[...], w_ref[...], (((1,), (1,)), ((), ())),
        preferred_element_type=jnp.float32,
    )
    hp_ref[0:N, :] = h * dinv
    hp_ref[N:NP, :] = jnp.zeros((NP - N, D), jnp.float32)


def _tc2_body(s_ref, hp_ref, degp_ref, w_ref, b_ref, o_ref):
    dinv = _dinv_from_parts(degp_ref)
    s = s_ref[0, 0:N, :] + s_ref[1, 0:N, :] + hp_ref[0:N, :]
    z = jnp.maximum(s * dinv + b_ref[...], 0.0)
    h2 = lax.dot_general(
        z, w_ref[...], (((1,), (1,)), ((), ())),
        preferred_element_type=jnp.float32,
    )
    o_ref[0:N, :] = h2 * dinv
    o_ref[N:NP, :] = jnp.zeros((NP - N, D), jnp.float32)


def _tc3_body(s_ref, hp_ref, degp_ref, b_ref, o_ref):
    dinv = _dinv_from_parts(degp_ref)
    s = s_ref[0, 0:N, :] + s_ref[1, 0:N, :] + hp_ref[0:N, :]
    o_ref[...] = s * dinv + b_ref[...]


def kernel(x, edge_index, W1, b1, W2, b2):
    E = edge_index.shape[1]
    K = -(-E // (NW * CH))
    EP = NW * K * CH
    pad = jnp.full((EP - E,), N, jnp.int32)
    src3 = jnp.concatenate([edge_index[0], pad]).reshape(NW, K, CH)
    dst3 = jnp.concatenate([edge_index[1], pad]).reshape(NW, K, CH)

    sc_deg = _make_sc_deg(K)
    sc_edge = _make_sc_edge(K)

    degp = sc_deg(dst3)

    h1p = pl.pallas_call(
        _tc1_body,
        out_shape=jax.ShapeDtypeStruct((NP, D), jnp.float32),
    )(x, W1, degp)

    s1 = sc_edge(h1p, src3, dst3)

    h2p = pl.pallas_call(
        _tc2_body,
        out_shape=jax.ShapeDtypeStruct((NP, D), jnp.float32),
    )(s1, h1p, degp, W2, b1.reshape(1, D))

    s2 = sc_edge(h2p, src3, dst3)

    out = pl.pallas_call(
        _tc3_body,
        out_shape=jax.ShapeDtypeStruct((N, D), jnp.float32),
    )(s2, h2p, degp, b2.reshape(1, D))

    return out


# speedup vs baseline: 12.6688x; 12.6688x over previous
import functools

import jax
import jax.numpy as jnp
from jax import lax
from jax.experimental import pallas as pl
from jax.experimental.pallas import tpu as pltpu
from jax.experimental.pallas import tpu_sc as plsc

N = 10000
D = 128
NP = 10240
NC = 2
NS = 16
NW = NC * NS
CH = 128
RPS = NP // NS

_MESH = plsc.VectorSubcoreMesh(core_axis_name="c", subcore_axis_name="s")


def _worker_id():
    return lax.axis_index("s") * NC + lax.axis_index("c")


def _fill_rows(ref, rows, value):
    cols = ref.shape[1]

    @pl.loop(0, rows)
    def _(r):
        @pl.loop(0, cols // 16)
        def _(cc):
            ref[r, pl.ds(cc * 16, 16)] = jnp.full((16,), value, jnp.float32)


def _zero_acc(acc_sh, zrow_v):
    sid = lax.axis_index("s")
    base = sid * RPS
    zr = zrow_v.shape[0]

    @pl.loop(0, RPS // zr)
    def _(t):
        pltpu.sync_copy(zrow_v, acc_sh.at[pl.ds(base + t * zr, zr)])


def _make_sc_deg(K):
    @functools.partial(
        pl.kernel,
        mesh=_MESH,
        out_type=jax.ShapeDtypeStruct((NC, NP, 16), jnp.float32),
        scratch_types=[
            pltpu.VMEM((K, CH), jnp.int32),
            pltpu.VMEM((CH, 16), jnp.float32),
            pltpu.VMEM_SHARED((NP, 16), jnp.float32),
        ],
    )
    def sc_deg(dst_hbm, out_hbm, idx_v, ones_v, acc_sh):
        cid = lax.axis_index("c")
        sid = lax.axis_index("s")
        wid = _worker_id()
        base = sid * RPS

        _fill_rows(ones_v, CH, 0.0)
        _zero_acc(acc_sh, ones_v)
        _fill_rows(ones_v, CH, 1.0)
        plsc.subcore_barrier()

        pltpu.sync_copy(dst_hbm.at[wid], idx_v)

        @pl.loop(0, K)
        def _(j):
            pltpu.sync_copy(ones_v, acc_sh.at[idx_v.at[j]], add=True)

        plsc.subcore_barrier()
        pltpu.sync_copy(
            acc_sh.at[pl.ds(base, RPS)], out_hbm.at[cid].at[pl.ds(base, RPS)]
        )

    return sc_deg


def _make_sc_edge(K):
    @functools.partial(
        pl.kernel,
        mesh=_MESH,
        out_type=jax.ShapeDtypeStruct((NC, NP, D), jnp.float32),
        scratch_types=[
            pltpu.VMEM((K, CH), jnp.int32),
            pltpu.VMEM((K, CH), jnp.int32),
            pltpu.VMEM((CH, D), jnp.float32),
            pltpu.VMEM_SHARED((NP, D), jnp.float32),
        ],
    )
    def sc_edge(tab_hbm, src_hbm, dst_hbm, out_hbm, si_v, di_v, rows_v,
                acc_sh):
        cid = lax.axis_index("c")
        sid = lax.axis_index("s")
        wid = _worker_id()
        base = sid * RPS

        _fill_rows(rows_v, CH, 0.0)
        _zero_acc(acc_sh, rows_v)
        plsc.subcore_barrier()

        pltpu.sync_copy(src_hbm.at[wid], si_v)
        pltpu.sync_copy(dst_hbm.at[wid], di_v)

        @pl.loop(0, K)
        def _(j):
            pltpu.sync_copy(tab_hbm.at[si_v.at[j]], rows_v)
            pltpu.sync_copy(rows_v, acc_sh.at[di_v.at[j]], add=True)

        plsc.subcore_barrier()
        pltpu.sync_copy(
            acc_sh.at[pl.ds(base, RPS)], out_hbm.at[cid].at[pl.ds(base, RPS)]
        )

    return sc_edge


def _dinv_from_parts(degp_ref):
    deg = 1.0 + degp_ref[0, 0:N, 0:1] + degp_ref[1, 0:N, 0:1]
    return lax.rsqrt(deg)


def _tc1_body(x_ref, w_ref, degp_ref, hp_ref):
    dinv = _dinv_from_parts(degp_ref)
    h = lax.dot_general(
        x_ref[...], w_ref[...], (((1,), (1,)), ((), ())),
        preferred_element_type=jnp.float32,
    )
    hp_ref[0:N, :] = h * dinv
    hp_ref[N:NP, :] = jnp.zeros((NP - N, D), jnp.float32)


def _tc2_body(s_ref, hp_ref, degp_ref, w_ref, b_ref, o_ref):
    dinv = _dinv_from_parts(degp_ref)
    s = s_ref[0, 0:N, :] + s_ref[1, 0:N, :] + hp_ref[0:N, :]
    z = jnp.maximum(s * dinv + b_ref[...], 0.0)
    h2 = lax.dot_general(
        z, w_ref[...], (((1,), (1,)), ((), ())),
        preferred_element_type=jnp.float32,
    )
    o_ref[0:N, :] = h2 * dinv
    o_ref[N:NP, :] = jnp.zeros((NP - N, D), jnp.float32)


def _tc3_body(s_ref, hp_ref, degp_ref, b_ref, o_ref):
    dinv = _dinv_from_parts(degp_ref)
    s = s_ref[0, 0:N, :] + s_ref[1, 0:N, :] + hp_ref[0:N, :]
    o_ref[...] = s * dinv + b_ref[...]


def kernel(x, edge_index, W1, b1, W2, b2):
    E = edge_index.shape[1]
    K = -(-E // (NW * CH))
    EP = NW * K * CH
    pad = jnp.full((EP - E,), N, jnp.int32)
    src3 = jnp.concatenate([edge_index[0], pad]).reshape(NW, K, CH)
    dst3 = jnp.concatenate([edge_index[1], pad]).reshape(NW, K, CH)

    sc_deg = _make_sc_deg(K)
    sc_edge = _make_sc_edge(K)

    degp = sc_deg(dst3)

    h1p = pl.pallas_call(
        _tc1_body,
        out_shape=jax.ShapeDtypeStruct((NP, D), jnp.float32),
    )(x, W1, degp)

    s1 = sc_edge(h1p, src3, dst3)

    h2p = pl.pallas_call(
        _tc2_body,
        out_shape=jax.ShapeDtypeStruct((NP, D), jnp.float32),
    )(s1, h1p, degp, W2, b1.reshape(1, D))

    s2 = sc_edge(h2p, src3, dst3)

    out = pl.pallas_call(
        _tc3_body,
        out_shape=jax.ShapeDtypeStruct((N, D), jnp.float32),
    )(s2, h2p, degp, b2.reshape(1, D))

    return out
